# baseline (speedup 1.0000x reference)
import functools

import jax
import jax.numpy as jnp
from jax.experimental import pallas as pl

B, T, D = 4, 2048, 1024
E, TOP_K = 16, 2
ROWS = B * T
BLOCK_ROWS = 512


def _router_block(x_ref, wt_ref, swt_ref, b_ref, sb_ref, ei_ref, eo_ref,
                  out_ref, idx_ref, z_ref):
    i = pl.program_id(0)
    n = pl.num_programs(0)

    ei = ei_ref[...]
    eo = eo_ref[...]
    ei = jnp.sign(ei) * jnp.sqrt(jnp.abs(ei))
    eo = jnp.sign(eo) * jnp.sqrt(jnp.abs(eo))
    noisy_wt = wt_ref[...] + swt_ref[...] * (ei * eo)
    noisy_b = b_ref[...] + sb_ref[...] * eo

    x = x_ref[...]
    logits = jnp.dot(x, noisy_wt,
                     preferred_element_type=jnp.float32) + noisy_b

    iota = jax.lax.broadcasted_iota(jnp.int32, logits.shape, 1)
    m1 = jnp.max(logits, axis=1, keepdims=True)
    i1 = jnp.min(jnp.where(logits == m1, iota, E), axis=1, keepdims=True)
    rest = jnp.where(iota == i1, -jnp.inf, logits)
    m2 = jnp.max(rest, axis=1, keepdims=True)
    i2 = jnp.min(jnp.where(rest == m2, iota, E), axis=1, keepdims=True)

    mask = (iota == i1) | (iota == i2)
    e2 = jnp.exp(m2 - m1)
    denom = 1.0 + e2
    probs = jnp.where(mask, jnp.exp(logits - m1) / denom, 0.0)
    out_ref[...] = probs
    idx_ref[...] = jnp.concatenate([i1, i2], axis=1)

    lse = m1 + jnp.log(denom)
    part = jnp.sum(lse * lse, keepdims=True).reshape(1, 1)

    @pl.when(i == 0)
    def _():
        z_ref[...] = jnp.zeros((1, 1), jnp.float32)

    z_ref[...] += part

    @pl.when(i == n - 1)
    def _():
        z_ref[...] = z_ref[...] / ROWS


@jax.jit
def kernel(mh_output, W, sigma_W, b, sigma_b, eps_in, eps_out):
    x = mh_output.reshape(ROWS, D)
    wt = W.T
    swt = sigma_W.T
    ei = eps_in.reshape(D, 1)
    eo = eps_out.reshape(1, E)
    b2 = b.reshape(1, E)
    sb2 = sigma_b.reshape(1, E)

    grid = (ROWS // BLOCK_ROWS,)
    out, idx, z = pl.pallas_call(
        _router_block,
        grid=grid,
        in_specs=[
            pl.BlockSpec((BLOCK_ROWS, D), lambda i: (i, 0)),
            pl.BlockSpec((D, E), lambda i: (0, 0)),
            pl.BlockSpec((D, E), lambda i: (0, 0)),
            pl.BlockSpec((1, E), lambda i: (0, 0)),
            pl.BlockSpec((1, E), lambda i: (0, 0)),
            pl.BlockSpec((D, 1), lambda i: (0, 0)),
            pl.BlockSpec((1, E), lambda i: (0, 0)),
        ],
        out_specs=[
            pl.BlockSpec((BLOCK_ROWS, E), lambda i: (i, 0)),
            pl.BlockSpec((BLOCK_ROWS, TOP_K), lambda i: (i, 0)),
            pl.BlockSpec((1, 1), lambda i: (0, 0)),
        ],
        out_shape=[
            jax.ShapeDtypeStruct((ROWS, E), jnp.float32),
            jax.ShapeDtypeStruct((ROWS, TOP_K), jnp.int32),
            jax.ShapeDtypeStruct((1, 1), jnp.float32),
        ],
    )(x, wt, swt, b2, sb2, ei, eo)

    router_output = out.reshape(B, T, E)
    indices = idx.reshape(B, T, TOP_K)
    z_loss = z[0, 0]
    return (router_output, indices, z_loss)
